# Initial kernel scaffold; baseline (speedup 1.0000x reference)
import jax
import jax.numpy as jnp
from jax.experimental import pallas as pl


def kernel(score, weight, target):
    raise NotImplementedError("write your pallas kernel here")


# speedup vs baseline: 95.0128x; 95.0128x over previous
import functools

import jax
import jax.numpy as jnp
from jax import lax
from jax.experimental import pallas as pl
from jax.experimental.pallas import tpu as pltpu
from jax.experimental.pallas import tpu_sc as plsc

_B, _C, _H, _W = 8, 6, 512, 512
_N = _B * _H * _W
_KRANK = 131072
_THR09_BITS = 0x3F666666


_BH = 64


def _ce_body(w_ref, s_ref, t_ref, loss_ref, pred_ref):
    s = s_ref[0]
    t = t_ref[0]
    m = jnp.max(s, axis=0)
    den = jnp.sum(jnp.exp(s - m[None]), axis=0)
    lse = m + jnp.log(den)
    st = jnp.zeros_like(m)
    w = jnp.zeros_like(m)
    for c in range(_C):
        sel = t == c
        st = jnp.where(sel, s[c], st)
        w = jnp.where(sel, w_ref[0, c], w)
    loss_ref[0] = w * (lse - st)
    pred_ref[0] = lax.bitcast_convert_type(jnp.exp(st - lse), jnp.int32)


def _run_ce(w8, score, target):
    grid = (_B, _H // _BH)
    return pl.pallas_call(
        _ce_body,
        grid=grid,
        in_specs=[
            pl.BlockSpec((1, 8), lambda b, h: (0, 0), memory_space=pltpu.SMEM),
            pl.BlockSpec((1, _C, _BH, _W), lambda b, h: (b, 0, h, 0)),
            pl.BlockSpec((1, _BH, _W), lambda b, h: (b, h, 0)),
        ],
        out_specs=[
            pl.BlockSpec((1, _BH, _W), lambda b, h: (b, h, 0)),
            pl.BlockSpec((1, _BH, _W), lambda b, h: (b, h, 0)),
        ],
        out_shape=[
            jax.ShapeDtypeStruct((_B, _H, _W), jnp.float32),
            jax.ShapeDtypeStruct((_B, _H, _W), jnp.int32),
        ],
    )(w8, score, target)


_NC, _NS, _L = 2, 16, 16
_NW = _NC * _NS
_PW = _N // _NW
_CH = 8192
_HB1 = 16384
_HB2 = 65536

def _zero_vmem(ref, n):
    def z(i, _):
        ref[pl.ds(i * _L, _L)] = jnp.zeros((_L,), jnp.int32)
        return 0
    lax.fori_loop(0, n // _L, z, 0)


def _reduce_tiles(hist, tmp, acc, shared, out_hbm, cid, sid, nb):
    seg = nb // _NS
    pltpu.sync_copy(hist, shared.at[sid])
    plsc.subcore_barrier()
    pltpu.sync_copy(shared.at[0, pl.ds(sid * seg, seg)], acc)
    for t in range(1, _NS):
        pltpu.sync_copy(shared.at[t, pl.ds(sid * seg, seg)], tmp)

        def add(j, _):
            acc[pl.ds(j * _L, _L)] = acc[pl.ds(j * _L, _L)] + tmp[pl.ds(j * _L, _L)]
            return 0
        lax.fori_loop(0, seg // _L, add, 0)
    pltpu.sync_copy(acc, out_hbm.at[cid, pl.ds(sid * seg, seg)])


@functools.lru_cache(maxsize=None)
def _sc_kernels():
    mesh = plsc.VectorSubcoreMesh(core_axis_name="c", subcore_axis_name="s",
                                  num_cores=_NC, num_subcores=_NS)

    @functools.partial(
        pl.kernel,
        out_type=jax.ShapeDtypeStruct((_NC, _HB1), jnp.int32),
        mesh=mesh,
        compiler_params=pltpu.CompilerParams(needs_layout_passes=False),
        scratch_types=[
            pltpu.VMEM((_CH,), jnp.int32),
            pltpu.VMEM((_HB1,), jnp.int32),
            pltpu.VMEM((_HB1 // _NS,), jnp.int32),
            pltpu.VMEM((_HB1 // _NS,), jnp.int32),
            pltpu.VMEM_SHARED((_NS, _HB1), jnp.int32),
        ],
    )
    def hist1(pred_hbm, out_hbm, buf, hist, tmp, acc, shared):
        cid = lax.axis_index("c")
        sid = lax.axis_index("s")
        base = (cid * _NS + sid) * _PW
        _zero_vmem(hist, _HB1)
        ones = jnp.ones((_L,), jnp.int32)

        def chunk(ci, _):
            pltpu.sync_copy(pred_hbm.at[pl.ds(base + ci * _CH, _CH)], buf)

            def inner(j, _):
                b = buf[pl.ds(j * _L, _L)] >> 16
                plsc.addupdate_scatter(hist, [b], ones)
                return 0
            lax.fori_loop(0, _CH // _L, inner, 0)
            return 0
        lax.fori_loop(0, _PW // _CH, chunk, 0)
        _reduce_tiles(hist, tmp, acc, shared, out_hbm, cid, sid, _HB1)

    @functools.partial(
        pl.kernel,
        out_type=jax.ShapeDtypeStruct((_NW, _HB2), jnp.int32),
        mesh=mesh,
        compiler_params=pltpu.CompilerParams(needs_layout_passes=False),
        scratch_types=[
            pltpu.VMEM((_CH,), jnp.int32),
            pltpu.VMEM((_HB2,), jnp.int32),
            pltpu.VMEM((_L,), jnp.int32),
        ],
    )
    def hist2(pred_hbm, sel_hbm, out_hbm, buf, hist, selbuf):
        cid = lax.axis_index("c")
        sid = lax.axis_index("s")
        base = (cid * _NS + sid) * _PW
        pltpu.sync_copy(sel_hbm.at[pl.ds(0, _L)], selbuf)
        bucketv = selbuf[...]
        _zero_vmem(hist, _HB2)
        ones = jnp.ones((_L,), jnp.int32)

        def chunk(ci, _):
            pltpu.sync_copy(pred_hbm.at[pl.ds(base + ci * _CH, _CH)], buf)

            def inner(j, _):
                bits = buf[pl.ds(j * _L, _L)]
                top = bits >> 16
                low = bits & 0xFFFF
                plsc.addupdate_scatter(hist, [low], ones, mask=top == bucketv)
                return 0
            lax.fori_loop(0, _CH // _L, inner, 0)
            return 0
        lax.fori_loop(0, _PW // _CH, chunk, 0)
        pltpu.sync_copy(hist, out_hbm.at[cid * _NS + sid])

    return hist1, hist2


def _cum_and_rank(h, side, kk):
    r = lax.broadcasted_iota(jnp.int32, (side, side), 0)
    c = lax.broadcasted_iota(jnp.int32, (side, side), 1)
    upper = (r <= c).astype(jnp.float32)
    strict_lower = (c < r).astype(jnp.float32)
    rowcum = jnp.dot(h, upper, preferred_element_type=jnp.float32)
    rowtot = rowcum[:, side - 1:side]
    prefix = jnp.dot(strict_lower, rowtot,
                     preferred_element_type=jnp.float32)
    cum = rowcum + prefix
    kf = jnp.asarray(kk + 1, jnp.float32)
    pos = jnp.sum((cum < kf).astype(jnp.int32))
    flat = r * side + c
    excl = jnp.sum(jnp.where(flat == pos, cum - h, 0.0)).astype(jnp.int32)
    return pos, excl


def _sel1_body(h_ref, out_ref):
    h = (h_ref[0] + h_ref[1]).astype(jnp.float32)
    bucket, cnt_below = _cum_and_rank(h, 128, _KRANK)
    row = lax.broadcasted_iota(jnp.int32, (8, 128), 0)
    out_ref[...] = jnp.where(row == 0, bucket,
                             jnp.where(row == 1, cnt_below, 0))


def _run_sel1(h1):
    return pl.pallas_call(
        _sel1_body,
        out_shape=jax.ShapeDtypeStruct((8, 128), jnp.int32),
    )(h1.reshape(2, 128, 128))


def _sel2_body(sel_ref, h_ref, thr_ref):
    hi = h_ref[0]
    for t in range(1, _NW):
        hi = hi + h_ref[t]
    h = hi.astype(jnp.float32)
    bucket = sel_ref[0, 0]
    cnt_below = sel_ref[1, 0]
    k2 = _KRANK - cnt_below
    low, _ = _cum_and_rank(h, 256, k2)
    thr_ref[0, 0] = jnp.maximum((bucket << 16) | low, _THR09_BITS)


def _run_sel2(sel, h2):
    return pl.pallas_call(
        _sel2_body,
        in_specs=[
            pl.BlockSpec(memory_space=pltpu.SMEM),
            pl.BlockSpec((_NW, 256, 256), lambda: (0, 0, 0)),
        ],
        out_specs=pl.BlockSpec(memory_space=pltpu.SMEM),
        out_shape=jax.ShapeDtypeStruct((1, 1), jnp.int32),
    )(sel, h2.reshape(_NW, 256, 256))


_RROWS, _RCOLS = 2048, 1024
_BR = 256
_RG = _RROWS // _BR


def _red_body(thr_ref, loss_ref, pred_ref, out_ref, acc_ref):
    i = pl.program_id(0)

    @pl.when(i == 0)
    def _():
        acc_ref[...] = jnp.zeros_like(acc_ref)

    thr = thr_ref[0, 0]
    keep = pred_ref[...] < thr
    acc_ref[0:1, :] += jnp.sum(jnp.where(keep, loss_ref[...], 0.0), axis=0,
                               keepdims=True)
    acc_ref[1:2, :] += jnp.sum(keep.astype(jnp.float32), axis=0, keepdims=True)

    @pl.when(i == _RG - 1)
    def _():
        s = jnp.sum(acc_ref[0:1, :])
        c = jnp.sum(acc_ref[1:2, :])
        out_ref[0, 0] = s / jnp.maximum(c, 1.0)


def _run_red(thr, loss2, pred2):
    return pl.pallas_call(
        _red_body,
        grid=(_RG,),
        in_specs=[
            pl.BlockSpec((1, 1), lambda i: (0, 0), memory_space=pltpu.SMEM),
            pl.BlockSpec((_BR, _RCOLS), lambda i: (i, 0)),
            pl.BlockSpec((_BR, _RCOLS), lambda i: (i, 0)),
        ],
        out_specs=pl.BlockSpec((1, 1), lambda i: (0, 0),
                               memory_space=pltpu.SMEM),
        out_shape=jax.ShapeDtypeStruct((1, 1), jnp.float32),
        scratch_shapes=[pltpu.VMEM((2, _RCOLS), jnp.float32)],
    )(thr, loss2, pred2)


def _fin_body(mean_ref, loss_ref, out_ref):
    out_ref[...] = loss_ref[...] + mean_ref[0, 0]


def _run_fin(mean, loss2):
    return pl.pallas_call(
        _fin_body,
        grid=(_RG,),
        in_specs=[
            pl.BlockSpec((1, 1), lambda i: (0, 0), memory_space=pltpu.SMEM),
            pl.BlockSpec((_BR, _RCOLS), lambda i: (i, 0)),
        ],
        out_specs=pl.BlockSpec((_BR, _RCOLS), lambda i: (i, 0)),
        out_shape=jax.ShapeDtypeStruct((_RROWS, _RCOLS), jnp.float32),
    )(mean, loss2)


def kernel(score, weight, target):
    w8 = jnp.pad(weight, (0, 8 - _C)).reshape(1, 8)
    loss3, pred3 = _run_ce(w8, score, target)
    pred1 = pred3.reshape(_N)
    loss2 = loss3.reshape(_RROWS, _RCOLS)
    pred2 = pred3.reshape(_RROWS, _RCOLS)
    hist1, hist2 = _sc_kernels()
    h1 = hist1(pred1)
    sel = _run_sel1(h1)
    h2 = hist2(pred1, sel.reshape(-1))
    thr = _run_sel2(sel, h2)
    mean = _run_red(thr, loss2, pred2)
    out = _run_fin(mean, loss2)
    return out.reshape(_N)


# speedup vs baseline: 131.8371x; 1.3876x over previous
import functools

import jax
import jax.numpy as jnp
from jax import lax
from jax.experimental import pallas as pl
from jax.experimental.pallas import tpu as pltpu
from jax.experimental.pallas import tpu_sc as plsc

_B, _C, _H, _W = 8, 6, 512, 512
_N = _B * _H * _W
_KRANK = 131072
_THR09_BITS = 0x3F666666
_RROWS, _RCOLS = 4096, 512


_BH = 64


def _ce_body(w_ref, s_ref, t_ref, loss_ref, pred_ref):
    s = s_ref[0]
    t = t_ref[...]
    m = jnp.max(s, axis=0)
    den = jnp.sum(jnp.exp(s - m[None]), axis=0)
    lse = m + jnp.log(den)
    st = jnp.zeros_like(m)
    w = jnp.zeros_like(m)
    for c in range(_C):
        sel = t == c
        st = jnp.where(sel, s[c], st)
        w = jnp.where(sel, w_ref[0, c], w)
    loss_ref[...] = w * (lse - st)
    pred_ref[...] = lax.bitcast_convert_type(jnp.exp(st - lse), jnp.int32)


def _run_ce(w8, score, target):
    grid = (_B, _H // _BH)
    return pl.pallas_call(
        _ce_body,
        grid=grid,
        in_specs=[
            pl.BlockSpec((1, 8), lambda b, h: (0, 0), memory_space=pltpu.SMEM),
            pl.BlockSpec((1, _C, _BH, _W), lambda b, h: (b, 0, h, 0)),
            pl.BlockSpec((_BH, _W), lambda b, h: (b * (_H // _BH) + h, 0)),
        ],
        out_specs=[
            pl.BlockSpec((_BH, _W), lambda b, h: (b * (_H // _BH) + h, 0)),
            pl.BlockSpec((_BH, _W), lambda b, h: (b * (_H // _BH) + h, 0)),
        ],
        out_shape=[
            jax.ShapeDtypeStruct((_RROWS, _RCOLS), jnp.float32),
            jax.ShapeDtypeStruct((_RROWS, _RCOLS), jnp.int32),
        ],
    )(w8, score, target.reshape(_RROWS, _RCOLS))


_NC, _NS, _L = 2, 16, 16
_NW = _NC * _NS
_WROWS = _RROWS // _NW
_CR = 16
_NCHUNK = _WROWS // _CR


def _zero_hist(ref, rows, cols):
    def z(r, _):
        for c in range(cols // _L):
            ref[r, pl.ds(c * _L, _L)] = jnp.zeros((_L,), jnp.int32)
        return 0
    lax.fori_loop(0, rows, z, 0)


def _stream_chunks(pred_hbm, row0, buf0, buf1, sem0, sem1, process):
    bufs = (buf0, buf1)
    sems = (sem0, sem1)
    copies = [None, None]
    copies[0] = pltpu.async_copy(
        pred_hbm.at[pl.ds(row0, _CR), :], bufs[0], sems[0])
    for ci in range(_NCHUNK):
        cur = ci % 2
        nxt = (ci + 1) % 2
        if ci + 1 < _NCHUNK:
            copies[nxt] = pltpu.async_copy(
                pred_hbm.at[pl.ds(row0 + (ci + 1) * _CR, _CR), :],
                bufs[nxt], sems[nxt])
        copies[cur].wait()
        process(bufs[cur])


def _reduce_tiles(hist, tmp, acc, shared, out_hbm, cid, sid, rows, cols):
    seg = rows // _NS
    pltpu.sync_copy(hist, shared.at[sid])
    plsc.subcore_barrier()
    pltpu.sync_copy(shared.at[0, pl.ds(sid * seg, seg), :], acc)
    for t in range(1, _NS):
        pltpu.sync_copy(shared.at[t, pl.ds(sid * seg, seg), :], tmp)

        def add(i, _):
            for c in range(cols // _L):
                sl = pl.ds(c * _L, _L)
                acc[i, sl] = acc[i, sl] + tmp[i, sl]
            return 0
        lax.fori_loop(0, seg, add, 0)
    pltpu.sync_copy(acc, out_hbm.at[cid, pl.ds(sid * seg, seg), :])


@functools.lru_cache(maxsize=None)
def _sc_kernels():
    mesh = plsc.VectorSubcoreMesh(core_axis_name="c", subcore_axis_name="s",
                                  num_cores=_NC, num_subcores=_NS)

    @functools.partial(
        pl.kernel,
        out_type=jax.ShapeDtypeStruct((_NC, 128, 128), jnp.int32),
        mesh=mesh,
        compiler_params=pltpu.CompilerParams(needs_layout_passes=False),
        scratch_types=[
            pltpu.VMEM((_CR, _RCOLS), jnp.int32),
            pltpu.VMEM((_CR, _RCOLS), jnp.int32),
            pltpu.VMEM((128, 128), jnp.int32),
            pltpu.VMEM((8, 128), jnp.int32),
            pltpu.VMEM((8, 128), jnp.int32),
            pltpu.VMEM_SHARED((_NS, 128, 128), jnp.int32),
            pltpu.SemaphoreType.DMA,
            pltpu.SemaphoreType.DMA,
        ],
    )
    def hist1(pred_hbm, out_hbm, buf0, buf1, hist, tmp, acc, shared,
              sem0, sem1):
        cid = lax.axis_index("c")
        sid = lax.axis_index("s")
        row0 = (cid * _NS + sid) * _WROWS
        _zero_hist(hist, 128, 128)
        ones = jnp.ones((_L,), jnp.int32)

        def process(buf):
            def rowfn(i, _):
                for j in range(_RCOLS // _L):
                    b = buf[i, pl.ds(j * _L, _L)] >> 16
                    plsc.addupdate_scatter(hist, [b >> 7, b & 127], ones)
                return 0
            lax.fori_loop(0, _CR, rowfn, 0)

        _stream_chunks(pred_hbm, row0, buf0, buf1, sem0, sem1, process)
        _reduce_tiles(hist, tmp, acc, shared, out_hbm, cid, sid, 128, 128)

    @functools.partial(
        pl.kernel,
        out_type=jax.ShapeDtypeStruct((_NW, 256, 256), jnp.int32),
        mesh=mesh,
        compiler_params=pltpu.CompilerParams(needs_layout_passes=False),
        scratch_types=[
            pltpu.VMEM((_CR, _RCOLS), jnp.int32),
            pltpu.VMEM((_CR, _RCOLS), jnp.int32),
            pltpu.VMEM((256, 256), jnp.int32),
            pltpu.VMEM((128,), jnp.int32),
            pltpu.SemaphoreType.DMA,
            pltpu.SemaphoreType.DMA,
        ],
    )
    def hist2(pred_hbm, sel_hbm, out_hbm, buf0, buf1, hist, selbuf,
              sem0, sem1):
        cid = lax.axis_index("c")
        sid = lax.axis_index("s")
        wid = cid * _NS + sid
        row0 = wid * _WROWS
        pltpu.sync_copy(sel_hbm.at[0], selbuf)
        bucketv = selbuf[pl.ds(0, _L)]
        _zero_hist(hist, 256, 256)
        ones = jnp.ones((_L,), jnp.int32)

        def process(buf):
            def rowfn(i, _):
                for j in range(_RCOLS // _L):
                    bits = buf[i, pl.ds(j * _L, _L)]
                    top = bits >> 16
                    low = bits & 0xFFFF
                    plsc.addupdate_scatter(hist, [low >> 8, low & 255], ones,
                                           mask=top == bucketv)
                return 0
            lax.fori_loop(0, _CR, rowfn, 0)

        _stream_chunks(pred_hbm, row0, buf0, buf1, sem0, sem1, process)
        pltpu.sync_copy(hist, out_hbm.at[wid])

    return hist1, hist2


def _cum_and_rank(h, side, kk):
    r = lax.broadcasted_iota(jnp.int32, (side, side), 0)
    c = lax.broadcasted_iota(jnp.int32, (side, side), 1)
    upper = (r <= c).astype(jnp.float32)
    strict_lower = (c < r).astype(jnp.float32)
    rowcum = jnp.dot(h, upper, preferred_element_type=jnp.float32)
    rowtot = rowcum[:, side - 1:side]
    prefix = jnp.dot(strict_lower, rowtot,
                     preferred_element_type=jnp.float32)
    cum = rowcum + prefix
    kf = jnp.asarray(kk + 1, jnp.float32)
    pos = jnp.sum((cum < kf).astype(jnp.int32))
    flat = r * side + c
    excl = jnp.sum(jnp.where(flat == pos, cum - h, 0.0)).astype(jnp.int32)
    return pos, excl


def _sel1_body(h_ref, out_ref):
    h = (h_ref[0] + h_ref[1]).astype(jnp.float32)
    bucket, cnt_below = _cum_and_rank(h, 128, _KRANK)
    row = lax.broadcasted_iota(jnp.int32, (8, 128), 0)
    out_ref[...] = jnp.where(row == 0, bucket,
                             jnp.where(row == 1, cnt_below, 0))


def _run_sel1(h1):
    return pl.pallas_call(
        _sel1_body,
        out_shape=jax.ShapeDtypeStruct((8, 128), jnp.int32),
    )(h1)


def _sel2_body(sel_ref, h_ref, thr_ref):
    hi = h_ref[0]
    for t in range(1, _NW):
        hi = hi + h_ref[t]
    h = hi.astype(jnp.float32)
    bucket = sel_ref[0, 0]
    cnt_below = sel_ref[1, 0]
    k2 = _KRANK - cnt_below
    low, _ = _cum_and_rank(h, 256, k2)
    thr_ref[0, 0] = jnp.maximum((bucket << 16) | low, _THR09_BITS)


def _run_sel2(sel, h2):
    return pl.pallas_call(
        _sel2_body,
        in_specs=[
            pl.BlockSpec(memory_space=pltpu.SMEM),
            pl.BlockSpec((_NW, 256, 256), lambda: (0, 0, 0)),
        ],
        out_specs=pl.BlockSpec(memory_space=pltpu.SMEM),
        out_shape=jax.ShapeDtypeStruct((1, 1), jnp.int32),
    )(sel, h2)


_BR = 512
_RG = _RROWS // _BR


def _red_body(thr_ref, loss_ref, pred_ref, out_ref, acc_ref):
    i = pl.program_id(0)

    @pl.when(i == 0)
    def _():
        acc_ref[...] = jnp.zeros_like(acc_ref)

    thr = thr_ref[0, 0]
    keep = pred_ref[...] < thr
    acc_ref[0:1, :] += jnp.sum(jnp.where(keep, loss_ref[...], 0.0), axis=0,
                               keepdims=True)
    acc_ref[1:2, :] += jnp.sum(keep.astype(jnp.float32), axis=0, keepdims=True)

    @pl.when(i == _RG - 1)
    def _():
        s = jnp.sum(acc_ref[0:1, :])
        c = jnp.sum(acc_ref[1:2, :])
        out_ref[0, 0] = s / jnp.maximum(c, 1.0)


def _run_red(thr, loss2, pred2):
    return pl.pallas_call(
        _red_body,
        grid=(_RG,),
        in_specs=[
            pl.BlockSpec((1, 1), lambda i: (0, 0), memory_space=pltpu.SMEM),
            pl.BlockSpec((_BR, _RCOLS), lambda i: (i, 0)),
            pl.BlockSpec((_BR, _RCOLS), lambda i: (i, 0)),
        ],
        out_specs=pl.BlockSpec((1, 1), lambda i: (0, 0),
                               memory_space=pltpu.SMEM),
        out_shape=jax.ShapeDtypeStruct((1, 1), jnp.float32),
        scratch_shapes=[pltpu.VMEM((2, _RCOLS), jnp.float32)],
    )(thr, loss2, pred2)


def _fin_body(mean_ref, loss_ref, out_ref):
    out_ref[...] = (loss_ref[...] + mean_ref[0, 0]).reshape(_BR * _RCOLS)


def _run_fin(mean, loss2):
    return pl.pallas_call(
        _fin_body,
        grid=(_RG,),
        in_specs=[
            pl.BlockSpec((1, 1), lambda i: (0, 0), memory_space=pltpu.SMEM),
            pl.BlockSpec((_BR, _RCOLS), lambda i: (i, 0)),
        ],
        out_specs=pl.BlockSpec((_BR * _RCOLS,), lambda i: (i,)),
        out_shape=jax.ShapeDtypeStruct((_N,), jnp.float32),
    )(mean, loss2)


def kernel(score, weight, target):
    w8 = jnp.pad(weight, (0, 8 - _C)).reshape(1, 8)
    loss2, pred2 = _run_ce(w8, score, target)
    hist1, hist2 = _sc_kernels()
    h1 = hist1(pred2)
    sel = _run_sel1(h1)
    h2 = hist2(pred2, sel)
    thr = _run_sel2(sel, h2)
    mean = _run_red(thr, loss2, pred2)
    return _run_fin(mean, loss2)


# speedup vs baseline: 131.9350x; 1.0007x over previous
import functools

import jax
import jax.numpy as jnp
from jax import lax
from jax.experimental import pallas as pl
from jax.experimental.pallas import tpu as pltpu
from jax.experimental.pallas import tpu_sc as plsc

_B, _C, _H, _W = 8, 6, 512, 512
_N = _B * _H * _W
_KRANK = 131072
_THR09_BITS = 0x3F666666
_RROWS, _RCOLS = 4096, 512


_BH = 64


def _ce_body(w_ref, s_ref, t_ref, loss_ref, pred_ref):
    s = s_ref[0]
    t = t_ref[...]
    m = jnp.max(s, axis=0)
    den = jnp.sum(jnp.exp(s - m[None]), axis=0)
    lse = m + jnp.log(den)
    st = jnp.zeros_like(m)
    w = jnp.zeros_like(m)
    for c in range(_C):
        sel = t == c
        st = jnp.where(sel, s[c], st)
        w = jnp.where(sel, w_ref[0, c], w)
    loss_ref[...] = w * (lse - st)
    pred_ref[...] = lax.bitcast_convert_type(jnp.exp(st - lse), jnp.int32)


def _run_ce(w8, score, target):
    grid = (_B, _H // _BH)
    return pl.pallas_call(
        _ce_body,
        grid=grid,
        in_specs=[
            pl.BlockSpec((1, 8), lambda b, h: (0, 0), memory_space=pltpu.SMEM),
            pl.BlockSpec((1, _C, _BH, _W), lambda b, h: (b, 0, h, 0)),
            pl.BlockSpec((_BH, _W), lambda b, h: (b * (_H // _BH) + h, 0)),
        ],
        out_specs=[
            pl.BlockSpec((_BH, _W), lambda b, h: (b * (_H // _BH) + h, 0)),
            pl.BlockSpec((_BH, _W), lambda b, h: (b * (_H // _BH) + h, 0)),
        ],
        out_shape=[
            jax.ShapeDtypeStruct((_RROWS, _RCOLS), jnp.float32),
            jax.ShapeDtypeStruct((_RROWS, _RCOLS), jnp.int32),
        ],
    )(w8, score, target.reshape(_RROWS, _RCOLS))


_NC, _NS, _L = 2, 16, 16
_NW = _NC * _NS
_WROWS = _RROWS // _NW
_CR = 16
_NCHUNK = _WROWS // _CR


def _zero_hist(ref, rows, cols):
    def z(r, _):
        for c in range(cols // _L):
            ref[r, pl.ds(c * _L, _L)] = jnp.zeros((_L,), jnp.int32)
        return 0
    lax.fori_loop(0, rows, z, 0)


def _stream_chunks(pred_hbm, row0, buf0, buf1, sem0, sem1, process):
    bufs = (buf0, buf1)
    sems = (sem0, sem1)
    copies = [None, None]
    copies[0] = pltpu.async_copy(
        pred_hbm.at[pl.ds(row0, _CR), :], bufs[0], sems[0])
    for ci in range(_NCHUNK):
        cur = ci % 2
        nxt = (ci + 1) % 2
        if ci + 1 < _NCHUNK:
            copies[nxt] = pltpu.async_copy(
                pred_hbm.at[pl.ds(row0 + (ci + 1) * _CR, _CR), :],
                bufs[nxt], sems[nxt])
        copies[cur].wait()
        process(bufs[cur])


def _reduce_tiles(hist, tmp, acc, shared, out_hbm, cid, sid, rows, cols):
    seg = rows // _NS
    pltpu.sync_copy(hist, shared.at[sid])
    plsc.subcore_barrier()
    pltpu.sync_copy(shared.at[0, pl.ds(sid * seg, seg), :], acc)
    for t in range(1, _NS):
        pltpu.sync_copy(shared.at[t, pl.ds(sid * seg, seg), :], tmp)

        def add(i, _):
            for c in range(cols // _L):
                sl = pl.ds(c * _L, _L)
                acc[i, sl] = acc[i, sl] + tmp[i, sl]
            return 0
        lax.fori_loop(0, seg, add, 0)
    pltpu.sync_copy(acc, out_hbm.at[cid, pl.ds(sid * seg, seg), :])


@functools.lru_cache(maxsize=None)
def _sc_kernels():
    mesh = plsc.VectorSubcoreMesh(core_axis_name="c", subcore_axis_name="s",
                                  num_cores=_NC, num_subcores=_NS)

    @functools.partial(
        pl.kernel,
        out_type=jax.ShapeDtypeStruct((_NC, 128, 128), jnp.int32),
        mesh=mesh,
        compiler_params=pltpu.CompilerParams(needs_layout_passes=False),
        scratch_types=[
            pltpu.VMEM((_CR, _RCOLS), jnp.int32),
            pltpu.VMEM((_CR, _RCOLS), jnp.int32),
            pltpu.VMEM((128, 128), jnp.int32),
            pltpu.VMEM((128, 128), jnp.int32),
            pltpu.VMEM((8, 128), jnp.int32),
            pltpu.VMEM((8, 128), jnp.int32),
            pltpu.VMEM_SHARED((_NS, 128, 128), jnp.int32),
            pltpu.SemaphoreType.DMA,
            pltpu.SemaphoreType.DMA,
        ],
    )
    def hist1(pred_hbm, out_hbm, buf0, buf1, hist, histb, tmp, acc, shared,
              sem0, sem1):
        cid = lax.axis_index("c")
        sid = lax.axis_index("s")
        row0 = (cid * _NS + sid) * _WROWS
        _zero_hist(hist, 128, 128)
        _zero_hist(histb, 128, 128)
        ones = jnp.ones((_L,), jnp.int32)

        def process(buf):
            def rowfn(i, _):
                for j in range(_RCOLS // _L):
                    b = buf[i, pl.ds(j * _L, _L)] >> 16
                    dst = hist if j % 2 == 0 else histb
                    plsc.addupdate_scatter(dst, [b >> 7, b & 127], ones)
                return 0
            lax.fori_loop(0, _CR, rowfn, 0)

        _stream_chunks(pred_hbm, row0, buf0, buf1, sem0, sem1, process)

        def merge(i, _):
            for c in range(128 // _L):
                sl = pl.ds(c * _L, _L)
                hist[i, sl] = hist[i, sl] + histb[i, sl]
            return 0
        lax.fori_loop(0, 128, merge, 0)
        _reduce_tiles(hist, tmp, acc, shared, out_hbm, cid, sid, 128, 128)

    @functools.partial(
        pl.kernel,
        out_type=jax.ShapeDtypeStruct((_NW, 256, 256), jnp.int32),
        mesh=mesh,
        compiler_params=pltpu.CompilerParams(needs_layout_passes=False),
        scratch_types=[
            pltpu.VMEM((_CR, _RCOLS), jnp.int32),
            pltpu.VMEM((_CR, _RCOLS), jnp.int32),
            pltpu.VMEM((256, 256), jnp.int32),
            pltpu.VMEM((128,), jnp.int32),
            pltpu.SemaphoreType.DMA,
            pltpu.SemaphoreType.DMA,
        ],
    )
    def hist2(pred_hbm, sel_hbm, out_hbm, buf0, buf1, hist, selbuf,
              sem0, sem1):
        cid = lax.axis_index("c")
        sid = lax.axis_index("s")
        wid = cid * _NS + sid
        row0 = wid * _WROWS
        pltpu.sync_copy(sel_hbm.at[0], selbuf)
        bucketv = selbuf[pl.ds(0, _L)]
        _zero_hist(hist, 256, 256)
        ones = jnp.ones((_L,), jnp.int32)

        def process(buf):
            def rowfn(i, _):
                for j in range(_RCOLS // _L):
                    bits = buf[i, pl.ds(j * _L, _L)]
                    top = bits >> 16
                    low = bits & 0xFFFF
                    plsc.addupdate_scatter(hist, [low >> 8, low & 255], ones,
                                           mask=top == bucketv)
                return 0
            lax.fori_loop(0, _CR, rowfn, 0)

        _stream_chunks(pred_hbm, row0, buf0, buf1, sem0, sem1, process)
        pltpu.sync_copy(hist, out_hbm.at[wid])

    return hist1, hist2


def _cum_and_rank(h, side, kk):
    r = lax.broadcasted_iota(jnp.int32, (side, side), 0)
    c = lax.broadcasted_iota(jnp.int32, (side, side), 1)
    upper = (r <= c).astype(jnp.float32)
    strict_lower = (c < r).astype(jnp.float32)
    rowcum = jnp.dot(h, upper, preferred_element_type=jnp.float32)
    rowtot = rowcum[:, side - 1:side]
    prefix = jnp.dot(strict_lower, rowtot,
                     preferred_element_type=jnp.float32)
    cum = rowcum + prefix
    kf = jnp.asarray(kk + 1, jnp.float32)
    pos = jnp.sum((cum < kf).astype(jnp.int32))
    flat = r * side + c
    excl = jnp.sum(jnp.where(flat == pos, cum - h, 0.0)).astype(jnp.int32)
    return pos, excl


def _sel1_body(h_ref, out_ref):
    h = (h_ref[0] + h_ref[1]).astype(jnp.float32)
    bucket, cnt_below = _cum_and_rank(h, 128, _KRANK)
    row = lax.broadcasted_iota(jnp.int32, (8, 128), 0)
    out_ref[...] = jnp.where(row == 0, bucket,
                             jnp.where(row == 1, cnt_below, 0))


def _run_sel1(h1):
    return pl.pallas_call(
        _sel1_body,
        out_shape=jax.ShapeDtypeStruct((8, 128), jnp.int32),
    )(h1)


_BR = 512
_RG = _RROWS // _BR


def _red_body(sel_ref, h_ref, loss_ref, pred_ref, out_ref, acc_ref, thr_ref):
    i = pl.program_id(0)

    @pl.when(i == 0)
    def _():
        acc_ref[...] = jnp.zeros_like(acc_ref)
        hi = h_ref[0]
        for t in range(1, _NW):
            hi = hi + h_ref[t]
        h = hi.astype(jnp.float32)
        bucket = sel_ref[0, 0]
        cnt_below = sel_ref[1, 0]
        k2 = _KRANK - cnt_below
        low, _ = _cum_and_rank(h, 256, k2)
        thr_ref[0, 0] = jnp.maximum((bucket << 16) | low, _THR09_BITS)

    thr = thr_ref[0, 0]
    keep = pred_ref[...] < thr
    acc_ref[0:1, :] += jnp.sum(jnp.where(keep, loss_ref[...], 0.0), axis=0,
                               keepdims=True)
    acc_ref[1:2, :] += jnp.sum(keep.astype(jnp.float32), axis=0, keepdims=True)

    @pl.when(i == _RG - 1)
    def _():
        s = jnp.sum(acc_ref[0:1, :])
        c = jnp.sum(acc_ref[1:2, :])
        out_ref[0, 0] = s / jnp.maximum(c, 1.0)


def _run_red(sel, h2, loss2, pred2):
    return pl.pallas_call(
        _red_body,
        grid=(_RG,),
        in_specs=[
            pl.BlockSpec(memory_space=pltpu.SMEM),
            pl.BlockSpec((_NW, 256, 256), lambda i: (0, 0, 0)),
            pl.BlockSpec((_BR, _RCOLS), lambda i: (i, 0)),
            pl.BlockSpec((_BR, _RCOLS), lambda i: (i, 0)),
        ],
        out_specs=pl.BlockSpec((1, 1), lambda i: (0, 0),
                               memory_space=pltpu.SMEM),
        out_shape=jax.ShapeDtypeStruct((1, 1), jnp.float32),
        scratch_shapes=[pltpu.VMEM((2, _RCOLS), jnp.float32),
                        pltpu.SMEM((1, 1), jnp.int32)],
    )(sel, h2, loss2, pred2)


def _fin_body(mean_ref, loss_ref, out_ref):
    out_ref[...] = (loss_ref[...] + mean_ref[0, 0]).reshape(_BR * _RCOLS)


def _run_fin(mean, loss2):
    return pl.pallas_call(
        _fin_body,
        grid=(_RG,),
        in_specs=[
            pl.BlockSpec((1, 1), lambda i: (0, 0), memory_space=pltpu.SMEM),
            pl.BlockSpec((_BR, _RCOLS), lambda i: (i, 0)),
        ],
        out_specs=pl.BlockSpec((_BR * _RCOLS,), lambda i: (i,)),
        out_shape=jax.ShapeDtypeStruct((_N,), jnp.float32),
    )(mean, loss2)


def kernel(score, weight, target):
    w8 = jnp.pad(weight, (0, 8 - _C)).reshape(1, 8)
    loss2, pred2 = _run_ce(w8, score, target)
    hist1, hist2 = _sc_kernels()
    h1 = hist1(pred2)
    sel = _run_sel1(h1)
    h2 = hist2(pred2, sel)
    mean = _run_red(sel, h2, loss2, pred2)
    return _run_fin(mean, loss2)
